# baseline (speedup 1.0000x reference)
import functools

import jax
import jax.numpy as jnp
from jax.experimental import pallas as pl
from jax.experimental.pallas import tpu as pltpu

_VMEM = 64 * 1024 * 1024
_BF = jnp.bfloat16
_F32 = jnp.float32


def _taps(kh_n, kw_n):
    return [(i, j) for i in range(kh_n) for j in range(kw_n)]


def _slab_conv(p, ibuf, w_ref, b_ref, wp, r, cin):
    for t, (kh, kw) in enumerate(_taps(3, 3)):
        ibuf[0:r, t * cin:(t + 1) * cin] = p[kw, kh * wp:kh * wp + r, :]
    d = jnp.dot(ibuf[0:r, 0:9 * cin], w_ref[...],
                preferred_element_type=_F32)
    return jnp.maximum(d + b_ref[...], 0.0).astype(_BF)


def _mask_cols(y, wp, ow):
    t = jax.lax.broadcasted_iota(jnp.int32, y.shape, 0) % wp
    return jnp.where(t < ow, y, jnp.zeros_like(y))


def _store3(dst, ym, wp):
    r = ym.shape[0]
    for k in range(3):
        dst[k, wp + 1 - k:wp + 1 - k + r, :] = ym


def _aud_body(a0_ref, w0_ref, b0_ref, w1_ref, b1_ref, w2_ref, b2_ref,
              w3_ref, b3_ref, w4_ref, b4_ref, w5_ref, b5_ref,
              o_ref, p1, p2, p3, sm, s5, ibuf):
    for g in range(2):
        _aud_one(a0_ref.at[g], w0_ref, b0_ref, w1_ref, b1_ref, w2_ref,
                 b2_ref, w3_ref, b3_ref, w4_ref, b4_ref, w5_ref, b5_ref,
                 o_ref.at[g], p1.at[g], p2.at[g], p3.at[g], sm.at[g],
                 s5.at[g], ibuf.at[g])


def _aud_one(a0_ref, w0_ref, b0_ref, w1_ref, b1_ref, w2_ref, b2_ref,
             w3_ref, b3_ref, w4_ref, b4_ref, w5_ref, b5_ref,
             o_ref, p1, p2, p3, sm, s5, ibuf):
    p1[:, 0:120, :] = jnp.zeros((3, 120, 128), _BF)
    p1[:, 1560:1792, :] = jnp.zeros((3, 232, 128), _BF)
    p2[...] = jnp.zeros_like(p2)
    p3[:, 0:65, :] = jnp.zeros((3, 65, 384), _BF)
    p3[:, 752:896, :] = jnp.zeros((3, 144, 384), _BF)

    y0 = jnp.dot(a0_ref[...], w0_ref[...], preferred_element_type=_F32)
    y0 = jnp.maximum(y0 + b0_ref[...], 0.0).astype(_BF)
    _store3(p1, _mask_cols(y0, 112, 99), 112)

    y1 = _slab_conv(p1, ibuf, w1_ref, b1_ref, 112, 1443, 128)

    m1 = jnp.maximum(jnp.maximum(y1[0:1219, :], y1[112:1331, :]),
                     y1[224:1443, :]).astype(_F32)
    sm[0, 0:1219, :] = m1[:, 0:128]
    sm[1, 0:1219, :] = m1[:, 128:256]
    for ph in range(11):
        row = None
        for dw in range(3):
            v = jnp.concatenate([sm[0, pl.ds(ph * 112 + dw, 49, 2), :],
                                 sm[1, pl.ds(ph * 112 + dw, 49, 2), :]],
                                axis=1)
            row = v if row is None else jnp.maximum(row, v)
        row = row.astype(_BF)
        for k in range(3):
            p2[k, (ph + 1) * 64 + 1 - k:(ph + 1) * 64 + 50 - k, :] = row

    y2 = _slab_conv(p2, ibuf, w2_ref, b2_ref, 64, 689, 256)
    _store3(p3, _mask_cols(y2, 64, 49), 64)

    y3 = _slab_conv(p3, ibuf, w3_ref, b3_ref, 64, 689, 384)
    _store3(p2, _mask_cols(y3, 64, 49), 64)

    y4 = _slab_conv(p2, ibuf, w4_ref, b4_ref, 64, 689, 256)

    m2 = jnp.maximum(jnp.maximum(y4[0:561, :], y4[64:625, :]),
                     y4[128:689, :]).astype(_F32)
    sm[0, 0:561, :] = m2[:, 0:128]
    sm[1, 0:561, :] = m2[:, 128:256]
    for ph in range(5):
        row = None
        for dw in range(3):
            v = jnp.concatenate([sm[0, pl.ds(2 * ph * 64 + dw, 24, 2), :],
                                 sm[1, pl.ds(2 * ph * 64 + dw, 24, 2), :]],
                                axis=1)
            row = v if row is None else jnp.maximum(row, v)
        s5[ph * 24:(ph + 1) * 24, :] = row.astype(_BF)

    y5 = None
    for t, (kh, kw) in enumerate(_taps(5, 4)):
        off = kh * 24 + kw
        d = jnp.dot(s5[off:off + 21, :], w5_ref[t * 256:(t + 1) * 256, :],
                    preferred_element_type=_F32)
        y5 = d if y5 is None else y5 + d
    y5 = jnp.maximum(y5 + b5_ref[...], 0.0).astype(_BF)
    o_ref[...] = y5


def _audio_stack(a0, ws):
    B = a0.shape[0]
    specs = [pl.BlockSpec((2, 1456, 16), lambda i: (i, 0, 0))]
    for w in ws:
        specs.append(pl.BlockSpec(w.shape, lambda i: (0, 0)))
    return pl.pallas_call(
        _aud_body,
        out_shape=jax.ShapeDtypeStruct((B, 21, 512), _BF),
        grid=(B // 2,),
        in_specs=specs,
        out_specs=pl.BlockSpec((2, 21, 512), lambda i: (i, 0, 0)),
        scratch_shapes=[
            pltpu.VMEM((2, 3, 1792, 128), _BF),
            pltpu.VMEM((2, 3, 896, 256), _BF),
            pltpu.VMEM((2, 3, 896, 384), _BF),
            pltpu.VMEM((2, 2, 1224, 128), _F32),
            pltpu.VMEM((2, 120, 256), _BF),
            pltpu.VMEM((2, 1456, 3456), _BF),
        ],
        compiler_params=pltpu.CompilerParams(
            dimension_semantics=("parallel",), vmem_limit_bytes=_VMEM),
    )(a0, *ws)


def _fc_body(a_ref, w_ref, b_ref, o_ref, acc_ref, *, relu, nk):
    if nk == 1:
        y = jnp.dot(a_ref[...], w_ref[...],
                    preferred_element_type=_F32) + b_ref[...]
        if relu:
            y = jnp.maximum(y, 0.0)
        o_ref[...] = y.astype(o_ref.dtype)
        return
    k = pl.program_id(1)

    @pl.when(k == 0)
    def _():
        acc_ref[...] = jnp.zeros_like(acc_ref)

    acc_ref[...] += jnp.dot(a_ref[...], w_ref[...],
                            preferred_element_type=_F32)

    @pl.when(k == nk - 1)
    def _():
        y = acc_ref[...] + b_ref[...]
        if relu:
            y = jnp.maximum(y, 0.0)
        o_ref[...] = y.astype(o_ref.dtype)


def _fc(a, w, b, relu, out_dtype, tn, tk):
    M, K = a.shape
    kp, np_ = w.shape
    gn, nk = np_ // tn, kp // tk
    return pl.pallas_call(
        functools.partial(_fc_body, relu=relu, nk=nk),
        out_shape=jax.ShapeDtypeStruct((M, np_), out_dtype),
        grid=(gn, nk),
        in_specs=[
            pl.BlockSpec((M, tk), lambda j, k: (0, k)),
            pl.BlockSpec((tk, tn), lambda j, k: (k, j)),
            pl.BlockSpec((1, tn), lambda j, k: (0, j)),
        ],
        out_specs=pl.BlockSpec((M, tn), lambda j, k: (0, j)),
        scratch_shapes=[pltpu.VMEM((M, tn), _F32)],
        compiler_params=pltpu.CompilerParams(
            dimension_semantics=("parallel", "arbitrary"),
            vmem_limit_bytes=_VMEM),
    )(a.astype(_BF), w, b)


def _vid_body(a_ref, wl_ref, bl_ref, w1_ref, b1_ref, o_ref):
    y = jnp.dot(a_ref[...], wl_ref[...],
                preferred_element_type=_F32) + bl_ref[...]
    y = jnp.maximum(y, 0.0).astype(_BF)
    m, r = 32, a_ref.shape[0]
    row = jax.lax.broadcasted_iota(jnp.int32, (m, r), 0)
    col = jax.lax.broadcasted_iota(jnp.int32, (m, r), 1)
    pool = jnp.where(col // 8 == row, 0.125, 0.0).astype(_BF)
    pooled = jnp.dot(pool, y, preferred_element_type=_F32).astype(_BF)
    h = jnp.dot(pooled, w1_ref[...], preferred_element_type=_F32) + b1_ref[...]
    o_ref[...] = jnp.maximum(h, 0.0).astype(o_ref.dtype)


def _heads_body(ha_ref, hv_ref, w2a_ref, b2a_ref, w2v_ref, b2v_ref,
                sa_ref, ta_ref, sv_ref, tv_ref, wla_ref, bla_ref,
                wlv_ref, blv_ref, wc1_ref, bc1_ref, wc2_ref, bc2_ref,
                fo_ref, vf_ref, af_ref, vc_ref, ac_ref):
    af = jnp.dot(ha_ref[...], w2a_ref[...],
                 preferred_element_type=_F32) + b2a_ref[...]
    vf = jnp.dot(hv_ref[...], w2v_ref[...],
                 preferred_element_type=_F32) + b2v_ref[...]
    af_ref[...] = af
    vf_ref[...] = vf
    abn = af * sa_ref[...] + ta_ref[...]
    vbn = vf * sv_ref[...] + tv_ref[...]
    ac_ref[...] = jnp.dot(abn, wla_ref[...],
                          preferred_element_type=_F32) + bla_ref[...]
    vc_ref[...] = jnp.dot(vbn, wlv_ref[...],
                          preferred_element_type=_F32) + blv_ref[...]
    dn = (((1,), (1,)), ((), ()))
    hh = (jax.lax.dot_general(vf, wc1_ref[:, 0:1024], dn,
                              preferred_element_type=_F32)
          + jax.lax.dot_general(af, wc1_ref[:, 1024:2048], dn,
                                preferred_element_type=_F32)
          + bc1_ref[...])
    hh = jnp.maximum(hh, 0.0)
    fo_ref[...] = jnp.dot(hh, wc2_ref[...],
                          preferred_element_type=_F32) + bc2_ref[...]


def _bn_fold(g, be, rm, rv):
    s = g / jnp.sqrt(rv + 1e-5)
    return s[None, :], (be - rm * s)[None, :]


def _padw(w_2xk):
    return jnp.pad(jnp.transpose(w_2xk), ((0, 0), (0, 126)))


def kernel(aud_conv0_w, aud_conv0_b, aud_conv1_w, aud_conv1_b, aud_conv2_w,
           aud_conv2_b, aud_conv3_w, aud_conv3_b, aud_conv4_w, aud_conv4_b,
           aud_conv5_w, aud_conv5_b, fcaud_fc1_w, fcaud_fc1_b, fcaud_fc2_w,
           fcaud_fc2_b, lip_conv_w, lip_conv_b, fclip_fc1_w, fclip_fc1_b,
           fclip_fc2_w, fclip_fc2_b, final_bn_lip_gamma, final_bn_lip_beta,
           final_bn_lip_rm, final_bn_lip_rv, final_bn_aud_gamma,
           final_bn_aud_beta, final_bn_aud_rm, final_bn_aud_rv,
           final_fc_lip_w, final_fc_lip_b, final_fc_aud_w, final_fc_aud_b,
           final_cls_w1, final_cls_b1, final_cls_w2, final_cls_b2,
           video, audio):
    B = audio.shape[0]
    H, W = audio.shape[3], audio.shape[4]

    x = audio.reshape(B, H, W)
    xp = jnp.pad(x, ((0, 0), (1, 2), (1, 14)))
    pats = [xp[:, i:i + H, j:j + 112] for i in range(3) for j in range(3)]
    a0 = jnp.stack(pats, axis=-1).astype(_BF)
    a0 = jnp.pad(a0, ((0, 0), (0, 0), (0, 0), (0, 7))).reshape(B, H * 112, 16)

    o5 = _audio_stack(a0, (
        aud_conv0_w[:16], aud_conv0_b, aud_conv1_w, aud_conv1_b,
        aud_conv2_w, aud_conv2_b, aud_conv3_w, aud_conv3_b,
        aud_conv4_w, aud_conv4_b, aud_conv5_w, aud_conv5_b))
    mid = o5.transpose(0, 2, 1).reshape(B, 512 * 21)

    ha = _fc(mid, fcaud_fc1_w, fcaud_fc1_b, relu=True,
             out_dtype=_BF, tn=2048, tk=1792)

    v = jnp.transpose(video[:, 0, :, :, :2, :2], (0, 2, 3, 4, 1))
    vp = jnp.pad(v, ((0, 0), (2, 2), (1, 0), (1, 0), (0, 0)))
    pv = [vp[:, kt:kt + 29:4] for kt in range(3)]
    av = jnp.stack(pv, axis=2).reshape(B * 8, 81).astype(_BF)
    av = jnp.pad(av, ((0, 0), (0, 47)))
    hv = pl.pallas_call(
        _vid_body,
        out_shape=jax.ShapeDtypeStruct((B, 4096), _BF),
        grid=(4,),
        in_specs=[
            pl.BlockSpec((B * 8, 128), lambda j: (0, 0)),
            pl.BlockSpec((128, 2048), lambda j: (0, 0)),
            pl.BlockSpec((1, 2048), lambda j: (0, 0)),
            pl.BlockSpec((2048, 1024), lambda j: (0, j)),
            pl.BlockSpec((1, 1024), lambda j: (0, j)),
        ],
        out_specs=pl.BlockSpec((B, 1024), lambda j: (0, j)),
        compiler_params=pltpu.CompilerParams(
            dimension_semantics=("parallel",), vmem_limit_bytes=_VMEM),
    )(av, lip_conv_w, lip_conv_b, fclip_fc1_w, fclip_fc1_b)

    sa, ta = _bn_fold(final_bn_aud_gamma, final_bn_aud_beta,
                      final_bn_aud_rm, final_bn_aud_rv)
    sv, tv = _bn_fold(final_bn_lip_gamma, final_bn_lip_beta,
                      final_bn_lip_rm, final_bn_lip_rv)
    outs = pl.pallas_call(
        _heads_body,
        out_shape=(
            jax.ShapeDtypeStruct((B, 128), _F32),
            jax.ShapeDtypeStruct((B, 1024), _F32),
            jax.ShapeDtypeStruct((B, 1024), _F32),
            jax.ShapeDtypeStruct((B, 128), _F32),
            jax.ShapeDtypeStruct((B, 128), _F32),
        ),
        compiler_params=pltpu.CompilerParams(vmem_limit_bytes=_VMEM),
    )(ha, hv, fcaud_fc2_w, fcaud_fc2_b, fclip_fc2_w, fclip_fc2_b,
      sa, ta, sv, tv,
      _padw(final_fc_aud_w), jnp.pad(final_fc_aud_b, (0, 126))[None, :],
      _padw(final_fc_lip_w), jnp.pad(final_fc_lip_b, (0, 126))[None, :],
      final_cls_w1, final_cls_b1[None, :],
      jnp.pad(jnp.transpose(final_cls_w2), ((0, 0), (0, 126))),
      jnp.pad(final_cls_b2, (0, 126))[None, :])

    fo, vid_feat, aud_feat, vc, ac = outs
    return (fo[:, :2], vid_feat, aud_feat, vc[:, :2], ac[:, :2])


# speedup vs baseline: 1.0130x; 1.0130x over previous
import functools

import jax
import jax.numpy as jnp
from jax.experimental import pallas as pl
from jax.experimental.pallas import tpu as pltpu

_VMEM = 64 * 1024 * 1024
_BF = jnp.bfloat16
_F32 = jnp.float32


def _taps(kh_n, kw_n):
    return [(i, j) for i in range(kh_n) for j in range(kw_n)]


def _slab_conv(p, ibuf, w_ref, b_ref, wp, r, cin):
    for t, (kh, kw) in enumerate(_taps(3, 3)):
        ibuf[0:r, t * cin:(t + 1) * cin] = p[kw, kh * wp:kh * wp + r, :]
    d = jnp.dot(ibuf[0:r, 0:9 * cin], w_ref[...],
                preferred_element_type=_F32)
    return jnp.maximum(d + b_ref[...], 0.0).astype(_BF)


def _mask_cols(y, wp, ow):
    t = jax.lax.broadcasted_iota(jnp.int32, y.shape, 0) % wp
    return jnp.where(t < ow, y, jnp.zeros_like(y))


def _store3(dst, ym, wp):
    r = ym.shape[0]
    for k in range(3):
        dst[k, wp + 1 - k:wp + 1 - k + r, :] = ym


def _aud_body(a0_ref, w0_ref, b0_ref, w1_ref, b1_ref, w2_ref, b2_ref,
              w3_ref, b3_ref, w4_ref, b4_ref, w5_ref, b5_ref,
              o_ref, p1, p2, p3, sm, s5, ibuf):
    for g in range(2):
        _aud_one(a0_ref.at[g], w0_ref, b0_ref, w1_ref, b1_ref, w2_ref,
                 b2_ref, w3_ref, b3_ref, w4_ref, b4_ref, w5_ref, b5_ref,
                 o_ref.at[g], p1.at[g], p2.at[g], p3.at[g], sm.at[g],
                 s5.at[g], ibuf.at[g])


def _aud_one(a0_ref, w0_ref, b0_ref, w1_ref, b1_ref, w2_ref, b2_ref,
             w3_ref, b3_ref, w4_ref, b4_ref, w5_ref, b5_ref,
             o_ref, p1, p2, p3, sm, s5, ibuf):
    p1[:, 0:120, :] = jnp.zeros((3, 120, 64), _BF)
    p1[:, 1560:1792, :] = jnp.zeros((3, 232, 64), _BF)
    p2[...] = jnp.zeros_like(p2)
    p3[:, 0:65, :] = jnp.zeros((3, 65, 384), _BF)
    p3[:, 752:896, :] = jnp.zeros((3, 144, 384), _BF)

    y0 = jnp.dot(a0_ref[...], w0_ref[...], preferred_element_type=_F32)
    y0 = jnp.maximum(y0 + b0_ref[...], 0.0).astype(_BF)
    _store3(p1, _mask_cols(y0, 112, 99), 112)

    y1 = _slab_conv(p1, ibuf, w1_ref, b1_ref, 112, 1443, 64)

    m1 = jnp.maximum(jnp.maximum(y1[0:1219, :], y1[112:1331, :]),
                     y1[224:1443, :]).astype(_F32)
    sm[0, 0:1219, :] = m1[:, 0:128]
    sm[1, 0:1219, :] = m1[:, 128:256]
    for ph in range(11):
        row = None
        for dw in range(3):
            v = jnp.concatenate([sm[0, pl.ds(ph * 112 + dw, 49, 2), :],
                                 sm[1, pl.ds(ph * 112 + dw, 49, 2), :]],
                                axis=1)
            row = v if row is None else jnp.maximum(row, v)
        row = row.astype(_BF)
        for k in range(3):
            p2[k, (ph + 1) * 64 + 1 - k:(ph + 1) * 64 + 50 - k, :] = row

    y2 = _slab_conv(p2, ibuf, w2_ref, b2_ref, 64, 689, 256)
    _store3(p3, _mask_cols(y2, 64, 49), 64)

    y3 = _slab_conv(p3, ibuf, w3_ref, b3_ref, 64, 689, 384)
    _store3(p2, _mask_cols(y3, 64, 49), 64)

    y4 = _slab_conv(p2, ibuf, w4_ref, b4_ref, 64, 689, 256)

    m2 = jnp.maximum(jnp.maximum(y4[0:561, :], y4[64:625, :]),
                     y4[128:689, :]).astype(_F32)
    sm[0, 0:561, :] = m2[:, 0:128]
    sm[1, 0:561, :] = m2[:, 128:256]
    for ph in range(5):
        row = None
        for dw in range(3):
            v = jnp.concatenate([sm[0, pl.ds(2 * ph * 64 + dw, 24, 2), :],
                                 sm[1, pl.ds(2 * ph * 64 + dw, 24, 2), :]],
                                axis=1)
            row = v if row is None else jnp.maximum(row, v)
        s5[ph * 24:(ph + 1) * 24, :] = row.astype(_BF)

    y5 = None
    for t, (kh, kw) in enumerate(_taps(5, 4)):
        off = kh * 24 + kw
        d = jnp.dot(s5[off:off + 21, :], w5_ref[t * 256:(t + 1) * 256, :],
                    preferred_element_type=_F32)
        y5 = d if y5 is None else y5 + d
    y5 = jnp.maximum(y5 + b5_ref[...], 0.0).astype(_BF)
    o_ref[...] = y5


def _audio_stack(a0, ws):
    B = a0.shape[0]
    specs = [pl.BlockSpec((2, 1456, 16), lambda i: (i, 0, 0))]
    for w in ws:
        specs.append(pl.BlockSpec(w.shape, lambda i: (0, 0)))
    return pl.pallas_call(
        _aud_body,
        out_shape=jax.ShapeDtypeStruct((B, 21, 512), _BF),
        grid=(B // 2,),
        in_specs=specs,
        out_specs=pl.BlockSpec((2, 21, 512), lambda i: (i, 0, 0)),
        scratch_shapes=[
            pltpu.VMEM((2, 3, 1792, 64), _BF),
            pltpu.VMEM((2, 3, 896, 256), _BF),
            pltpu.VMEM((2, 3, 896, 384), _BF),
            pltpu.VMEM((2, 2, 1224, 128), _F32),
            pltpu.VMEM((2, 120, 256), _BF),
            pltpu.VMEM((2, 1456, 3456), _BF),
        ],
        compiler_params=pltpu.CompilerParams(
            dimension_semantics=("parallel",), vmem_limit_bytes=_VMEM),
    )(a0, *ws)


def _fc_body(a_ref, w_ref, b_ref, o_ref, acc_ref, *, relu, nk):
    if nk == 1:
        y = jnp.dot(a_ref[...], w_ref[...],
                    preferred_element_type=_F32) + b_ref[...]
        if relu:
            y = jnp.maximum(y, 0.0)
        o_ref[...] = y.astype(o_ref.dtype)
        return
    k = pl.program_id(1)

    @pl.when(k == 0)
    def _():
        acc_ref[...] = jnp.zeros_like(acc_ref)

    acc_ref[...] += jnp.dot(a_ref[...], w_ref[...],
                            preferred_element_type=_F32)

    @pl.when(k == nk - 1)
    def _():
        y = acc_ref[...] + b_ref[...]
        if relu:
            y = jnp.maximum(y, 0.0)
        o_ref[...] = y.astype(o_ref.dtype)


def _fc(a, w, b, relu, out_dtype, tn, tk):
    M, K = a.shape
    kp, np_ = w.shape
    gn, nk = np_ // tn, kp // tk
    return pl.pallas_call(
        functools.partial(_fc_body, relu=relu, nk=nk),
        out_shape=jax.ShapeDtypeStruct((M, np_), out_dtype),
        grid=(gn, nk),
        in_specs=[
            pl.BlockSpec((M, tk), lambda j, k: (0, k)),
            pl.BlockSpec((tk, tn), lambda j, k: (k, j)),
            pl.BlockSpec((1, tn), lambda j, k: (0, j)),
        ],
        out_specs=pl.BlockSpec((M, tn), lambda j, k: (0, j)),
        scratch_shapes=[pltpu.VMEM((M, tn), _F32)],
        compiler_params=pltpu.CompilerParams(
            dimension_semantics=("parallel", "arbitrary"),
            vmem_limit_bytes=_VMEM),
    )(a.astype(_BF), w, b)


def _vid_body(a_ref, wl_ref, bl_ref, w1_ref, b1_ref, o_ref):
    y = jnp.dot(a_ref[...], wl_ref[...],
                preferred_element_type=_F32) + bl_ref[...]
    y = jnp.maximum(y, 0.0).astype(_BF)
    m, r = 32, a_ref.shape[0]
    row = jax.lax.broadcasted_iota(jnp.int32, (m, r), 0)
    col = jax.lax.broadcasted_iota(jnp.int32, (m, r), 1)
    pool = jnp.where(col // 8 == row, 0.125, 0.0).astype(_BF)
    pooled = jnp.dot(pool, y, preferred_element_type=_F32).astype(_BF)
    h = jnp.dot(pooled, w1_ref[...], preferred_element_type=_F32) + b1_ref[...]
    o_ref[...] = jnp.maximum(h, 0.0).astype(o_ref.dtype)


def _heads_body(ha_ref, hv_ref, w2a_ref, b2a_ref, w2v_ref, b2v_ref,
                sa_ref, ta_ref, sv_ref, tv_ref, wla_ref, bla_ref,
                wlv_ref, blv_ref, wc1_ref, bc1_ref, wc2_ref, bc2_ref,
                fo_ref, vf_ref, af_ref, vc_ref, ac_ref):
    af = jnp.dot(ha_ref[...], w2a_ref[...],
                 preferred_element_type=_F32) + b2a_ref[...]
    vf = jnp.dot(hv_ref[...], w2v_ref[...],
                 preferred_element_type=_F32) + b2v_ref[...]
    af_ref[...] = af
    vf_ref[...] = vf
    abn = af * sa_ref[...] + ta_ref[...]
    vbn = vf * sv_ref[...] + tv_ref[...]
    ac_ref[...] = jnp.dot(abn, wla_ref[...],
                          preferred_element_type=_F32) + bla_ref[...]
    vc_ref[...] = jnp.dot(vbn, wlv_ref[...],
                          preferred_element_type=_F32) + blv_ref[...]
    dn = (((1,), (1,)), ((), ()))
    hh = (jax.lax.dot_general(vf, wc1_ref[:, 0:1024], dn,
                              preferred_element_type=_F32)
          + jax.lax.dot_general(af, wc1_ref[:, 1024:2048], dn,
                                preferred_element_type=_F32)
          + bc1_ref[...])
    hh = jnp.maximum(hh, 0.0)
    fo_ref[...] = jnp.dot(hh, wc2_ref[...],
                          preferred_element_type=_F32) + bc2_ref[...]


def _bn_fold(g, be, rm, rv):
    s = g / jnp.sqrt(rv + 1e-5)
    return s[None, :], (be - rm * s)[None, :]


def _padw(w_2xk):
    return jnp.pad(jnp.transpose(w_2xk), ((0, 0), (0, 126)))


def kernel(aud_conv0_w, aud_conv0_b, aud_conv1_w, aud_conv1_b, aud_conv2_w,
           aud_conv2_b, aud_conv3_w, aud_conv3_b, aud_conv4_w, aud_conv4_b,
           aud_conv5_w, aud_conv5_b, fcaud_fc1_w, fcaud_fc1_b, fcaud_fc2_w,
           fcaud_fc2_b, lip_conv_w, lip_conv_b, fclip_fc1_w, fclip_fc1_b,
           fclip_fc2_w, fclip_fc2_b, final_bn_lip_gamma, final_bn_lip_beta,
           final_bn_lip_rm, final_bn_lip_rv, final_bn_aud_gamma,
           final_bn_aud_beta, final_bn_aud_rm, final_bn_aud_rv,
           final_fc_lip_w, final_fc_lip_b, final_fc_aud_w, final_fc_aud_b,
           final_cls_w1, final_cls_b1, final_cls_w2, final_cls_b2,
           video, audio):
    B = audio.shape[0]
    H, W = audio.shape[3], audio.shape[4]

    x = audio.reshape(B, H, W)
    xp = jnp.pad(x, ((0, 0), (1, 2), (1, 14)))
    pats = [xp[:, i:i + H, j:j + 112] for i in range(3) for j in range(3)]
    a0 = jnp.stack(pats, axis=-1).astype(_BF)
    a0 = jnp.pad(a0, ((0, 0), (0, 0), (0, 0), (0, 7))).reshape(B, H * 112, 16)

    w1p = jnp.concatenate([aud_conv1_w[t * 128:t * 128 + 64]
                           for t in range(9)], axis=0)
    o5 = _audio_stack(a0, (
        aud_conv0_w[:16, :64], aud_conv0_b[:, :64], w1p, aud_conv1_b,
        aud_conv2_w, aud_conv2_b, aud_conv3_w, aud_conv3_b,
        aud_conv4_w, aud_conv4_b, aud_conv5_w, aud_conv5_b))
    mid = o5.transpose(0, 2, 1).reshape(B, 512 * 21)

    ha = _fc(mid, fcaud_fc1_w, fcaud_fc1_b, relu=True,
             out_dtype=_BF, tn=2048, tk=1792)

    v = jnp.transpose(video[:, 0, :, :, :2, :2], (0, 2, 3, 4, 1))
    vp = jnp.pad(v, ((0, 0), (2, 2), (1, 0), (1, 0), (0, 0)))
    pv = [vp[:, kt:kt + 29:4] for kt in range(3)]
    av = jnp.stack(pv, axis=2).reshape(B * 8, 81).astype(_BF)
    av = jnp.pad(av, ((0, 0), (0, 47)))
    hv = pl.pallas_call(
        _vid_body,
        out_shape=jax.ShapeDtypeStruct((B, 4096), _BF),
        grid=(4,),
        in_specs=[
            pl.BlockSpec((B * 8, 128), lambda j: (0, 0)),
            pl.BlockSpec((128, 2048), lambda j: (0, 0)),
            pl.BlockSpec((1, 2048), lambda j: (0, 0)),
            pl.BlockSpec((2048, 1024), lambda j: (0, j)),
            pl.BlockSpec((1, 1024), lambda j: (0, j)),
        ],
        out_specs=pl.BlockSpec((B, 1024), lambda j: (0, j)),
        compiler_params=pltpu.CompilerParams(
            dimension_semantics=("parallel",), vmem_limit_bytes=_VMEM),
    )(av, lip_conv_w, lip_conv_b, fclip_fc1_w, fclip_fc1_b)

    sa, ta = _bn_fold(final_bn_aud_gamma, final_bn_aud_beta,
                      final_bn_aud_rm, final_bn_aud_rv)
    sv, tv = _bn_fold(final_bn_lip_gamma, final_bn_lip_beta,
                      final_bn_lip_rm, final_bn_lip_rv)
    outs = pl.pallas_call(
        _heads_body,
        out_shape=(
            jax.ShapeDtypeStruct((B, 128), _F32),
            jax.ShapeDtypeStruct((B, 1024), _F32),
            jax.ShapeDtypeStruct((B, 1024), _F32),
            jax.ShapeDtypeStruct((B, 128), _F32),
            jax.ShapeDtypeStruct((B, 128), _F32),
        ),
        compiler_params=pltpu.CompilerParams(vmem_limit_bytes=_VMEM),
    )(ha, hv, fcaud_fc2_w, fcaud_fc2_b, fclip_fc2_w, fclip_fc2_b,
      sa, ta, sv, tv,
      _padw(final_fc_aud_w), jnp.pad(final_fc_aud_b, (0, 126))[None, :],
      _padw(final_fc_lip_w), jnp.pad(final_fc_lip_b, (0, 126))[None, :],
      final_cls_w1, final_cls_b1[None, :],
      jnp.pad(jnp.transpose(final_cls_w2), ((0, 0), (0, 126))),
      jnp.pad(final_cls_b2, (0, 126))[None, :])

    fo, vid_feat, aud_feat, vc, ac = outs
    return (fo[:, :2], vid_feat, aud_feat, vc[:, :2], ac[:, :2])


# speedup vs baseline: 1.1729x; 1.1579x over previous
import functools

import jax
import jax.numpy as jnp
from jax.experimental import pallas as pl
from jax.experimental.pallas import tpu as pltpu

_VMEM = 64 * 1024 * 1024
_BF = jnp.bfloat16
_F32 = jnp.float32


def _taps(kh_n, kw_n):
    return [(i, j) for i in range(kh_n) for j in range(kw_n)]


def _slab_conv(p, ibuf, w_ref, b_ref, wp, r, cin):
    for t, (kh, kw) in enumerate(_taps(3, 3)):
        ibuf[0:r, t * cin:(t + 1) * cin] = p[kw, kh * wp:kh * wp + r, :]
    d = jnp.dot(ibuf[0:r, 0:9 * cin], w_ref[...],
                preferred_element_type=_F32)
    return jnp.maximum(d + b_ref[...], 0.0).astype(_BF)


def _mask_cols(y, wp, ow):
    t = jax.lax.broadcasted_iota(jnp.int32, y.shape, 0) % wp
    return jnp.where(t < ow, y, jnp.zeros_like(y))


def _store3(dst, ym, wp):
    r = ym.shape[0]
    for k in range(3):
        dst[k, wp + 1 - k:wp + 1 - k + r, :] = ym


def _aud_body(a0_ref, w0_ref, b0_ref, w1_ref, b1_ref, w2_ref, b2_ref,
              w3_ref, b3_ref, w4_ref, b4_ref, w5_ref, b5_ref, o_ref, *scr):
    imgs = []
    for g in range(2):
        imgs.append({'a0': a0_ref.at[g], 'o': o_ref.at[g],
                     'p1': scr[g * 6 + 0], 'p2': scr[g * 6 + 1],
                     'p3': scr[g * 6 + 2], 'sm': scr[g * 6 + 3],
                     's5': scr[g * 6 + 4], 'ib': scr[g * 6 + 5]})

    def cp(s, pk, wp, r, cin):
        for t, (kh, kw) in enumerate(_taps(3, 3)):
            s['ib'][0:r, t * cin:(t + 1) * cin] = \
                s[pk][kw, kh * wp:kh * wp + r, :]

    def dt(s, w_ref, b_ref, r, k):
        d = jnp.dot(s['ib'][0:r, 0:k], w_ref[...],
                    preferred_element_type=_F32)
        return jnp.maximum(d + b_ref[...], 0.0).astype(_BF)

    def st0(s):
        p1, p2, p3 = s['p1'], s['p2'], s['p3']
        p1[:, 0:120, :] = jnp.zeros((3, 120, 64), _BF)
        p1[:, 1560:1792, :] = jnp.zeros((3, 232, 64), _BF)
        p2[...] = jnp.zeros_like(p2)
        p3[:, 0:65, :] = jnp.zeros((3, 65, 384), _BF)
        p3[:, 752:896, :] = jnp.zeros((3, 144, 384), _BF)
        y0 = jnp.dot(s['a0'][...], w0_ref[...], preferred_element_type=_F32)
        y0 = jnp.maximum(y0 + b0_ref[...], 0.0).astype(_BF)
        _store3(p1, _mask_cols(y0, 112, 99), 112)

    def st1(s):
        cp(s, 'p1', 112, 1443, 64)

    def st2(s):
        s['y1'] = dt(s, w1_ref, b1_ref, 1443, 576)

    def st3(s):
        y1, sm, p2 = s['y1'], s['sm'], s['p2']
        m1 = jnp.maximum(jnp.maximum(y1[0:1219, :], y1[112:1331, :]),
                         y1[224:1443, :]).astype(_F32)
        sm[0, 0:1219, :] = m1[:, 0:128]
        sm[1, 0:1219, :] = m1[:, 128:256]
        for ph in range(11):
            row = None
            for dw in range(3):
                v = jnp.concatenate(
                    [sm[0, pl.ds(ph * 112 + dw, 49, 2), :],
                     sm[1, pl.ds(ph * 112 + dw, 49, 2), :]], axis=1)
                row = v if row is None else jnp.maximum(row, v)
            row = row.astype(_BF)
            for k in range(3):
                p2[k, (ph + 1) * 64 + 1 - k:(ph + 1) * 64 + 50 - k, :] = row

    def st4(s):
        cp(s, 'p2', 64, 689, 256)

    def st5(s):
        _store3(s['p3'], _mask_cols(dt(s, w2_ref, b2_ref, 689, 2304),
                                    64, 49), 64)

    def st6(s):
        cp(s, 'p3', 64, 689, 384)

    def st7(s):
        _store3(s['p2'], _mask_cols(dt(s, w3_ref, b3_ref, 689, 3456),
                                    64, 49), 64)

    def st8(s):
        cp(s, 'p2', 64, 689, 256)

    def st9(s):
        s['y4'] = dt(s, w4_ref, b4_ref, 689, 2304)

    def st10(s):
        y4, sm, s5 = s['y4'], s['sm'], s['s5']
        m2 = jnp.maximum(jnp.maximum(y4[0:561, :], y4[64:625, :]),
                         y4[128:689, :]).astype(_F32)
        sm[0, 0:561, :] = m2[:, 0:128]
        sm[1, 0:561, :] = m2[:, 128:256]
        for ph in range(5):
            row = None
            for dw in range(3):
                v = jnp.concatenate(
                    [sm[0, pl.ds(2 * ph * 64 + dw, 24, 2), :],
                     sm[1, pl.ds(2 * ph * 64 + dw, 24, 2), :]], axis=1)
                row = v if row is None else jnp.maximum(row, v)
            s5[ph * 24:(ph + 1) * 24, :] = row.astype(_BF)

    def st11(s):
        y5 = None
        for t, (kh, kw) in enumerate(_taps(5, 4)):
            off = kh * 24 + kw
            d = jnp.dot(s['s5'][off:off + 21, :],
                        w5_ref[t * 256:(t + 1) * 256, :],
                        preferred_element_type=_F32)
            y5 = d if y5 is None else y5 + d
        s['o'][...] = jnp.maximum(y5 + b5_ref[...], 0.0).astype(_BF)

    stages = [st0, st1, st2, st3, st4, st5, st6, st7, st8, st9, st10, st11]
    stages[0](imgs[0])
    for i in range(1, 12):
        stages[i](imgs[0])
        stages[i - 1](imgs[1])
    stages[11](imgs[1])


def _audio_stack(a0, ws):
    B = a0.shape[0]
    specs = [pl.BlockSpec((2, 1456, 16), lambda i: (i, 0, 0))]
    for w in ws:
        specs.append(pl.BlockSpec(w.shape, lambda i: (0, 0)))
    return pl.pallas_call(
        _aud_body,
        out_shape=jax.ShapeDtypeStruct((B, 21, 512), _BF),
        grid=(B // 2,),
        in_specs=specs,
        out_specs=pl.BlockSpec((2, 21, 512), lambda i: (i, 0, 0)),
        scratch_shapes=[s for _ in range(2) for s in (
            pltpu.VMEM((3, 1792, 64), _BF),
            pltpu.VMEM((3, 896, 256), _BF),
            pltpu.VMEM((3, 896, 384), _BF),
            pltpu.VMEM((2, 1224, 128), _F32),
            pltpu.VMEM((120, 256), _BF),
            pltpu.VMEM((1456, 3456), _BF),
        )],
        compiler_params=pltpu.CompilerParams(
            dimension_semantics=("parallel",), vmem_limit_bytes=_VMEM),
    )(a0, *ws)


def _fc_body(a_ref, w_ref, b_ref, o_ref, acc_ref, *, relu, nk):
    if nk == 1:
        y = jnp.dot(a_ref[...], w_ref[...],
                    preferred_element_type=_F32) + b_ref[...]
        if relu:
            y = jnp.maximum(y, 0.0)
        o_ref[...] = y.astype(o_ref.dtype)
        return
    k = pl.program_id(1)

    @pl.when(k == 0)
    def _():
        acc_ref[...] = jnp.zeros_like(acc_ref)

    acc_ref[...] += jnp.dot(a_ref[...], w_ref[...],
                            preferred_element_type=_F32)

    @pl.when(k == nk - 1)
    def _():
        y = acc_ref[...] + b_ref[...]
        if relu:
            y = jnp.maximum(y, 0.0)
        o_ref[...] = y.astype(o_ref.dtype)


def _fc(a, w, b, relu, out_dtype, tn, tk):
    M, K = a.shape
    kp, np_ = w.shape
    gn, nk = np_ // tn, kp // tk
    return pl.pallas_call(
        functools.partial(_fc_body, relu=relu, nk=nk),
        out_shape=jax.ShapeDtypeStruct((M, np_), out_dtype),
        grid=(gn, nk),
        in_specs=[
            pl.BlockSpec((M, tk), lambda j, k: (0, k)),
            pl.BlockSpec((tk, tn), lambda j, k: (k, j)),
            pl.BlockSpec((1, tn), lambda j, k: (0, j)),
        ],
        out_specs=pl.BlockSpec((M, tn), lambda j, k: (0, j)),
        scratch_shapes=[pltpu.VMEM((M, tn), _F32)],
        compiler_params=pltpu.CompilerParams(
            dimension_semantics=("parallel", "arbitrary"),
            vmem_limit_bytes=_VMEM),
    )(a.astype(_BF), w, b)


def _vid_body(a_ref, wl_ref, bl_ref, w1_ref, b1_ref, o_ref):
    y = jnp.dot(a_ref[...], wl_ref[...],
                preferred_element_type=_F32) + bl_ref[...]
    y = jnp.maximum(y, 0.0).astype(_BF)
    m, r = 32, a_ref.shape[0]
    row = jax.lax.broadcasted_iota(jnp.int32, (m, r), 0)
    col = jax.lax.broadcasted_iota(jnp.int32, (m, r), 1)
    pool = jnp.where(col // 8 == row, 0.125, 0.0).astype(_BF)
    pooled = jnp.dot(pool, y, preferred_element_type=_F32).astype(_BF)
    h = jnp.dot(pooled, w1_ref[...], preferred_element_type=_F32) + b1_ref[...]
    o_ref[...] = jnp.maximum(h, 0.0).astype(o_ref.dtype)


def _heads_body(ha_ref, hv_ref, w2a_ref, b2a_ref, w2v_ref, b2v_ref,
                sa_ref, ta_ref, sv_ref, tv_ref, wla_ref, bla_ref,
                wlv_ref, blv_ref, wc1_ref, bc1_ref, wc2_ref, bc2_ref,
                fo_ref, vf_ref, af_ref, vc_ref, ac_ref):
    af = jnp.dot(ha_ref[...], w2a_ref[...],
                 preferred_element_type=_F32) + b2a_ref[...]
    vf = jnp.dot(hv_ref[...], w2v_ref[...],
                 preferred_element_type=_F32) + b2v_ref[...]
    af_ref[...] = af
    vf_ref[...] = vf
    abn = af * sa_ref[...] + ta_ref[...]
    vbn = vf * sv_ref[...] + tv_ref[...]
    ac_ref[...] = jnp.dot(abn, wla_ref[...],
                          preferred_element_type=_F32) + bla_ref[...]
    vc_ref[...] = jnp.dot(vbn, wlv_ref[...],
                          preferred_element_type=_F32) + blv_ref[...]
    dn = (((1,), (1,)), ((), ()))
    hh = (jax.lax.dot_general(vf, wc1_ref[:, 0:1024], dn,
                              preferred_element_type=_F32)
          + jax.lax.dot_general(af, wc1_ref[:, 1024:2048], dn,
                                preferred_element_type=_F32)
          + bc1_ref[...])
    hh = jnp.maximum(hh, 0.0)
    fo_ref[...] = jnp.dot(hh, wc2_ref[...],
                          preferred_element_type=_F32) + bc2_ref[...]


def _bn_fold(g, be, rm, rv):
    s = g / jnp.sqrt(rv + 1e-5)
    return s[None, :], (be - rm * s)[None, :]


def _padw(w_2xk):
    return jnp.pad(jnp.transpose(w_2xk), ((0, 0), (0, 126)))


def kernel(aud_conv0_w, aud_conv0_b, aud_conv1_w, aud_conv1_b, aud_conv2_w,
           aud_conv2_b, aud_conv3_w, aud_conv3_b, aud_conv4_w, aud_conv4_b,
           aud_conv5_w, aud_conv5_b, fcaud_fc1_w, fcaud_fc1_b, fcaud_fc2_w,
           fcaud_fc2_b, lip_conv_w, lip_conv_b, fclip_fc1_w, fclip_fc1_b,
           fclip_fc2_w, fclip_fc2_b, final_bn_lip_gamma, final_bn_lip_beta,
           final_bn_lip_rm, final_bn_lip_rv, final_bn_aud_gamma,
           final_bn_aud_beta, final_bn_aud_rm, final_bn_aud_rv,
           final_fc_lip_w, final_fc_lip_b, final_fc_aud_w, final_fc_aud_b,
           final_cls_w1, final_cls_b1, final_cls_w2, final_cls_b2,
           video, audio):
    B = audio.shape[0]
    H, W = audio.shape[3], audio.shape[4]

    x = audio.reshape(B, H, W)
    xp = jnp.pad(x, ((0, 0), (1, 2), (1, 14)))
    pats = [xp[:, i:i + H, j:j + 112] for i in range(3) for j in range(3)]
    a0 = jnp.stack(pats, axis=-1).astype(_BF)
    a0 = jnp.pad(a0, ((0, 0), (0, 0), (0, 0), (0, 7))).reshape(B, H * 112, 16)

    w1p = jnp.concatenate([aud_conv1_w[t * 128:t * 128 + 64]
                           for t in range(9)], axis=0)
    o5 = _audio_stack(a0, (
        aud_conv0_w[:16, :64], aud_conv0_b[:, :64], w1p, aud_conv1_b,
        aud_conv2_w, aud_conv2_b, aud_conv3_w, aud_conv3_b,
        aud_conv4_w, aud_conv4_b, aud_conv5_w, aud_conv5_b))
    mid = o5.transpose(0, 2, 1).reshape(B, 512 * 21)

    ha = _fc(mid, fcaud_fc1_w, fcaud_fc1_b, relu=True,
             out_dtype=_BF, tn=2048, tk=1792)

    v = jnp.transpose(video[:, 0, :, :, :2, :2], (0, 2, 3, 4, 1))
    vp = jnp.pad(v, ((0, 0), (2, 2), (1, 0), (1, 0), (0, 0)))
    pv = [vp[:, kt:kt + 29:4] for kt in range(3)]
    av = jnp.stack(pv, axis=2).reshape(B * 8, 81).astype(_BF)
    av = jnp.pad(av, ((0, 0), (0, 47)))
    hv = pl.pallas_call(
        _vid_body,
        out_shape=jax.ShapeDtypeStruct((B, 4096), _BF),
        grid=(4,),
        in_specs=[
            pl.BlockSpec((B * 8, 128), lambda j: (0, 0)),
            pl.BlockSpec((128, 2048), lambda j: (0, 0)),
            pl.BlockSpec((1, 2048), lambda j: (0, 0)),
            pl.BlockSpec((2048, 1024), lambda j: (0, j)),
            pl.BlockSpec((1, 1024), lambda j: (0, j)),
        ],
        out_specs=pl.BlockSpec((B, 1024), lambda j: (0, j)),
        compiler_params=pltpu.CompilerParams(
            dimension_semantics=("parallel",), vmem_limit_bytes=_VMEM),
    )(av, lip_conv_w, lip_conv_b, fclip_fc1_w, fclip_fc1_b)

    sa, ta = _bn_fold(final_bn_aud_gamma, final_bn_aud_beta,
                      final_bn_aud_rm, final_bn_aud_rv)
    sv, tv = _bn_fold(final_bn_lip_gamma, final_bn_lip_beta,
                      final_bn_lip_rm, final_bn_lip_rv)
    outs = pl.pallas_call(
        _heads_body,
        out_shape=(
            jax.ShapeDtypeStruct((B, 128), _F32),
            jax.ShapeDtypeStruct((B, 1024), _F32),
            jax.ShapeDtypeStruct((B, 1024), _F32),
            jax.ShapeDtypeStruct((B, 128), _F32),
            jax.ShapeDtypeStruct((B, 128), _F32),
        ),
        compiler_params=pltpu.CompilerParams(vmem_limit_bytes=_VMEM),
    )(ha, hv, fcaud_fc2_w, fcaud_fc2_b, fclip_fc2_w, fclip_fc2_b,
      sa, ta, sv, tv,
      _padw(final_fc_aud_w), jnp.pad(final_fc_aud_b, (0, 126))[None, :],
      _padw(final_fc_lip_w), jnp.pad(final_fc_lip_b, (0, 126))[None, :],
      final_cls_w1, final_cls_b1[None, :],
      jnp.pad(jnp.transpose(final_cls_w2), ((0, 0), (0, 126))),
      jnp.pad(final_cls_b2, (0, 126))[None, :])

    fo, vid_feat, aud_feat, vc, ac = outs
    return (fo[:, :2], vid_feat, aud_feat, vc[:, :2], ac[:, :2])


# speedup vs baseline: 1.2550x; 1.0700x over previous
import functools

import jax
import jax.numpy as jnp
from jax.experimental import pallas as pl
from jax.experimental.pallas import tpu as pltpu

_VMEM = 64 * 1024 * 1024
_BF = jnp.bfloat16
_F32 = jnp.float32


def _taps(kh_n, kw_n):
    return [(i, j) for i in range(kh_n) for j in range(kw_n)]


def _slab_conv(p, ibuf, w_ref, b_ref, wp, r, cin):
    for t, (kh, kw) in enumerate(_taps(3, 3)):
        ibuf[0:r, t * cin:(t + 1) * cin] = p[kw, kh * wp:kh * wp + r, :]
    d = jnp.dot(ibuf[0:r, 0:9 * cin], w_ref[...],
                preferred_element_type=_F32)
    return jnp.maximum(d + b_ref[...], 0.0).astype(_BF)


def _mask_cols(y, wp, ow):
    t = jax.lax.broadcasted_iota(jnp.int32, y.shape, 0) % wp
    return jnp.where(t < ow, y, jnp.zeros_like(y))


def _store3(dst, ym, wp):
    r = ym.shape[0]
    for k in range(3):
        dst[k, wp + 1 - k:wp + 1 - k + r, :] = ym


def _aud_body(a0_ref, w0_ref, b0_ref, w1_ref, b1_ref, w2_ref, b2_ref,
              w3_ref, b3_ref, w4_ref, b4_ref, w5_ref, b5_ref, o_ref, *scr):
    NS = 7
    imgs = []
    for g in range(4):
        imgs.append({'a0': a0_ref.at[g], 'o': o_ref.at[g],
                     'p1': scr[g * NS + 0], 'p2': scr[g * NS + 1],
                     'p3': scr[g * NS + 2], 'sm': scr[g * NS + 3],
                     's5': scr[g * NS + 4], 'ib1': scr[g * NS + 5],
                     'ib': scr[g * NS + 6]})

    def cp(s, pk, cin, tap_lo, tap_hi, lane0):
        for t in range(tap_lo, tap_hi):
            kh, kw = t // 3, t % 3
            s['ib'][0:689, lane0 + (t - tap_lo) * cin:
                    lane0 + (t - tap_lo + 1) * cin] = \
                s[pk][kw, kh * 64:kh * 64 + 689, :]

    def st0(s):
        p1, p2, p3 = s['p1'], s['p2'], s['p3']
        p1[:, 0:120, :] = jnp.zeros((3, 120, 64), _BF)
        p1[:, 1560:1792, :] = jnp.zeros((3, 232, 64), _BF)
        p2[...] = jnp.zeros_like(p2)
        p3[:, 0:65, :] = jnp.zeros((3, 65, 384), _BF)
        p3[:, 752:896, :] = jnp.zeros((3, 144, 384), _BF)
        y0 = jnp.dot(s['a0'][...], w0_ref[...], preferred_element_type=_F32)
        y0 = jnp.maximum(y0 + b0_ref[...], 0.0).astype(_BF)
        _store3(p1, _mask_cols(y0, 112, 99), 112)

    def st1(s):
        for t, (kh, kw) in enumerate(_taps(3, 3)):
            s['ib1'][0:1443, t * 64:(t + 1) * 64] = \
                s['p1'][kw, kh * 112:kh * 112 + 1443, :]

    def st2(s):
        d = jnp.dot(s['ib1'][0:1443, :], w1_ref[...],
                    preferred_element_type=_F32)
        s['y1'] = jnp.maximum(d + b1_ref[...], 0.0).astype(_BF)

    def st3(s):
        y1, sm, p2 = s['y1'], s['sm'], s['p2']
        m1 = jnp.maximum(jnp.maximum(y1[0:1219, :], y1[112:1331, :]),
                         y1[224:1443, :]).astype(_F32)
        sm[0, 0:1219, :] = m1[:, 0:128]
        sm[1, 0:1219, :] = m1[:, 128:256]
        for ph in range(11):
            row = None
            for dw in range(3):
                v = jnp.concatenate(
                    [sm[0, pl.ds(ph * 112 + dw, 49, 2), :],
                     sm[1, pl.ds(ph * 112 + dw, 49, 2), :]], axis=1)
                row = v if row is None else jnp.maximum(row, v)
            row = row.astype(_BF)
            for k in range(3):
                p2[k, (ph + 1) * 64 + 1 - k:(ph + 1) * 64 + 50 - k, :] = row

    def st4(s):
        cp(s, 'p2', 256, 0, 9, 0)

    def st5(s):
        d = jnp.dot(s['ib'][0:689, 0:2304], w2_ref[...],
                    preferred_element_type=_F32)
        y2 = jnp.maximum(d + b2_ref[...], 0.0).astype(_BF)
        _store3(s['p3'], _mask_cols(y2, 64, 49), 64)

    def st6(s):
        cp(s, 'p3', 384, 0, 6, 0)

    def st7(s):
        s['d1'] = jnp.dot(s['ib'][0:689, 0:2304], w3_ref[0:2304, :],
                          preferred_element_type=_F32)

    def st8(s):
        cp(s, 'p3', 384, 6, 9, 0)

    def st9(s):
        d = s['d1'] + jnp.dot(s['ib'][0:689, 0:1152], w3_ref[2304:3456, :],
                              preferred_element_type=_F32)
        y3 = jnp.maximum(d + b3_ref[...], 0.0).astype(_BF)
        _store3(s['p2'], _mask_cols(y3, 64, 49), 64)

    def st10(s):
        cp(s, 'p2', 256, 0, 9, 0)

    def st11(s):
        d = jnp.dot(s['ib'][0:689, 0:2304], w4_ref[...],
                    preferred_element_type=_F32)
        s['y4'] = jnp.maximum(d + b4_ref[...], 0.0).astype(_BF)

    def st12(s):
        y4, sm, s5 = s['y4'], s['sm'], s['s5']
        m2 = jnp.maximum(jnp.maximum(y4[0:561, :], y4[64:625, :]),
                         y4[128:689, :]).astype(_F32)
        sm[0, 0:561, :] = m2[:, 0:128]
        sm[1, 0:561, :] = m2[:, 128:256]
        for ph in range(5):
            row = None
            for dw in range(3):
                v = jnp.concatenate(
                    [sm[0, pl.ds(2 * ph * 64 + dw, 24, 2), :],
                     sm[1, pl.ds(2 * ph * 64 + dw, 24, 2), :]], axis=1)
                row = v if row is None else jnp.maximum(row, v)
            s5[ph * 24:(ph + 1) * 24, :] = row.astype(_BF)

    def st13(s):
        y5 = None
        for t, (kh, kw) in enumerate(_taps(5, 4)):
            off = kh * 24 + kw
            d = jnp.dot(s['s5'][off:off + 21, :],
                        w5_ref[t * 256:(t + 1) * 256, :],
                        preferred_element_type=_F32)
            y5 = d if y5 is None else y5 + d
        s['o'][...] = jnp.maximum(y5 + b5_ref[...], 0.0).astype(_BF)

    stages = [st0, st1, st2, st3, st4, st5, st6, st7, st8, st9, st10, st11,
              st12, st13]
    for i in range(len(stages) + 3):
        for g in range(4):
            j = i - g
            if 0 <= j < len(stages):
                stages[j](imgs[g])


def _audio_stack(a0, ws):
    B = a0.shape[0]
    specs = [pl.BlockSpec((4, 1456, 16), lambda i: (i, 0, 0))]
    for w in ws:
        specs.append(pl.BlockSpec(w.shape, lambda i: (0, 0)))
    return pl.pallas_call(
        _aud_body,
        out_shape=jax.ShapeDtypeStruct((B, 21, 512), _BF),
        grid=(B // 4,),
        in_specs=specs,
        out_specs=pl.BlockSpec((4, 21, 512), lambda i: (i, 0, 0)),
        scratch_shapes=[s for _ in range(4) for s in (
            pltpu.VMEM((3, 1792, 64), _BF),
            pltpu.VMEM((3, 896, 256), _BF),
            pltpu.VMEM((3, 896, 384), _BF),
            pltpu.VMEM((2, 1224, 128), _F32),
            pltpu.VMEM((120, 256), _BF),
            pltpu.VMEM((1456, 576), _BF),
            pltpu.VMEM((704, 2304), _BF),
        )],
        compiler_params=pltpu.CompilerParams(
            dimension_semantics=("parallel",), vmem_limit_bytes=_VMEM),
    )(a0, *ws)


def _fc_body(a_ref, w_ref, b_ref, o_ref, acc_ref, *, relu, nk):
    if nk == 1:
        y = jnp.dot(a_ref[...], w_ref[...],
                    preferred_element_type=_F32) + b_ref[...]
        if relu:
            y = jnp.maximum(y, 0.0)
        o_ref[...] = y.astype(o_ref.dtype)
        return
    k = pl.program_id(1)

    @pl.when(k == 0)
    def _():
        acc_ref[...] = jnp.zeros_like(acc_ref)

    acc_ref[...] += jnp.dot(a_ref[...], w_ref[...],
                            preferred_element_type=_F32)

    @pl.when(k == nk - 1)
    def _():
        y = acc_ref[...] + b_ref[...]
        if relu:
            y = jnp.maximum(y, 0.0)
        o_ref[...] = y.astype(o_ref.dtype)


def _fc(a, w, b, relu, out_dtype, tn, tk):
    M, K = a.shape
    kp, np_ = w.shape
    gn, nk = np_ // tn, kp // tk
    return pl.pallas_call(
        functools.partial(_fc_body, relu=relu, nk=nk),
        out_shape=jax.ShapeDtypeStruct((M, np_), out_dtype),
        grid=(gn, nk),
        in_specs=[
            pl.BlockSpec((M, tk), lambda j, k: (0, k)),
            pl.BlockSpec((tk, tn), lambda j, k: (k, j)),
            pl.BlockSpec((1, tn), lambda j, k: (0, j)),
        ],
        out_specs=pl.BlockSpec((M, tn), lambda j, k: (0, j)),
        scratch_shapes=[pltpu.VMEM((M, tn), _F32)],
        compiler_params=pltpu.CompilerParams(
            dimension_semantics=("parallel", "arbitrary"),
            vmem_limit_bytes=_VMEM),
    )(a.astype(_BF), w, b)


def _vid_body(a_ref, wl_ref, bl_ref, w1_ref, b1_ref, o_ref):
    y = jnp.dot(a_ref[...], wl_ref[...],
                preferred_element_type=_F32) + bl_ref[...]
    y = jnp.maximum(y, 0.0).astype(_BF)
    m, r = 32, a_ref.shape[0]
    row = jax.lax.broadcasted_iota(jnp.int32, (m, r), 0)
    col = jax.lax.broadcasted_iota(jnp.int32, (m, r), 1)
    pool = jnp.where(col // 8 == row, 0.125, 0.0).astype(_BF)
    pooled = jnp.dot(pool, y, preferred_element_type=_F32).astype(_BF)
    h = jnp.dot(pooled, w1_ref[...], preferred_element_type=_F32) + b1_ref[...]
    o_ref[...] = jnp.maximum(h, 0.0).astype(o_ref.dtype)


def _heads_body(ha_ref, hv_ref, w2a_ref, b2a_ref, w2v_ref, b2v_ref,
                sa_ref, ta_ref, sv_ref, tv_ref, wla_ref, bla_ref,
                wlv_ref, blv_ref, wc1_ref, bc1_ref, wc2_ref, bc2_ref,
                fo_ref, vf_ref, af_ref, vc_ref, ac_ref):
    af = jnp.dot(ha_ref[...], w2a_ref[...],
                 preferred_element_type=_F32) + b2a_ref[...]
    vf = jnp.dot(hv_ref[...], w2v_ref[...],
                 preferred_element_type=_F32) + b2v_ref[...]
    af_ref[...] = af
    vf_ref[...] = vf
    abn = af * sa_ref[...] + ta_ref[...]
    vbn = vf * sv_ref[...] + tv_ref[...]
    ac_ref[...] = jnp.dot(abn, wla_ref[...],
                          preferred_element_type=_F32) + bla_ref[...]
    vc_ref[...] = jnp.dot(vbn, wlv_ref[...],
                          preferred_element_type=_F32) + blv_ref[...]
    dn = (((1,), (1,)), ((), ()))
    hh = (jax.lax.dot_general(vf, wc1_ref[:, 0:1024], dn,
                              preferred_element_type=_F32)
          + jax.lax.dot_general(af, wc1_ref[:, 1024:2048], dn,
                                preferred_element_type=_F32)
          + bc1_ref[...])
    hh = jnp.maximum(hh, 0.0)
    fo_ref[...] = jnp.dot(hh, wc2_ref[...],
                          preferred_element_type=_F32) + bc2_ref[...]


def _bn_fold(g, be, rm, rv):
    s = g / jnp.sqrt(rv + 1e-5)
    return s[None, :], (be - rm * s)[None, :]


def _padw(w_2xk):
    return jnp.pad(jnp.transpose(w_2xk), ((0, 0), (0, 126)))


def kernel(aud_conv0_w, aud_conv0_b, aud_conv1_w, aud_conv1_b, aud_conv2_w,
           aud_conv2_b, aud_conv3_w, aud_conv3_b, aud_conv4_w, aud_conv4_b,
           aud_conv5_w, aud_conv5_b, fcaud_fc1_w, fcaud_fc1_b, fcaud_fc2_w,
           fcaud_fc2_b, lip_conv_w, lip_conv_b, fclip_fc1_w, fclip_fc1_b,
           fclip_fc2_w, fclip_fc2_b, final_bn_lip_gamma, final_bn_lip_beta,
           final_bn_lip_rm, final_bn_lip_rv, final_bn_aud_gamma,
           final_bn_aud_beta, final_bn_aud_rm, final_bn_aud_rv,
           final_fc_lip_w, final_fc_lip_b, final_fc_aud_w, final_fc_aud_b,
           final_cls_w1, final_cls_b1, final_cls_w2, final_cls_b2,
           video, audio):
    B = audio.shape[0]
    H, W = audio.shape[3], audio.shape[4]

    x = audio.reshape(B, H, W)
    xp = jnp.pad(x, ((0, 0), (1, 2), (1, 14)))
    pats = [xp[:, i:i + H, j:j + 112] for i in range(3) for j in range(3)]
    a0 = jnp.stack(pats, axis=-1).astype(_BF)
    a0 = jnp.pad(a0, ((0, 0), (0, 0), (0, 0), (0, 7))).reshape(B, H * 112, 16)

    w1p = jnp.concatenate([aud_conv1_w[t * 128:t * 128 + 64]
                           for t in range(9)], axis=0)
    o5 = _audio_stack(a0, (
        aud_conv0_w[:16, :64], aud_conv0_b[:, :64], w1p, aud_conv1_b,
        aud_conv2_w, aud_conv2_b, aud_conv3_w, aud_conv3_b,
        aud_conv4_w, aud_conv4_b, aud_conv5_w, aud_conv5_b))
    mid = o5.transpose(0, 2, 1).reshape(B, 512 * 21)

    ha = _fc(mid, fcaud_fc1_w, fcaud_fc1_b, relu=True,
             out_dtype=_BF, tn=2048, tk=1792)

    v = jnp.transpose(video[:, 0, :, :, :2, :2], (0, 2, 3, 4, 1))
    vp = jnp.pad(v, ((0, 0), (2, 2), (1, 0), (1, 0), (0, 0)))
    pv = [vp[:, kt:kt + 29:4] for kt in range(3)]
    av = jnp.stack(pv, axis=2).reshape(B * 8, 81).astype(_BF)
    av = jnp.pad(av, ((0, 0), (0, 47)))
    hv = pl.pallas_call(
        _vid_body,
        out_shape=jax.ShapeDtypeStruct((B, 4096), _BF),
        grid=(4,),
        in_specs=[
            pl.BlockSpec((B * 8, 128), lambda j: (0, 0)),
            pl.BlockSpec((128, 2048), lambda j: (0, 0)),
            pl.BlockSpec((1, 2048), lambda j: (0, 0)),
            pl.BlockSpec((2048, 1024), lambda j: (0, j)),
            pl.BlockSpec((1, 1024), lambda j: (0, j)),
        ],
        out_specs=pl.BlockSpec((B, 1024), lambda j: (0, j)),
        compiler_params=pltpu.CompilerParams(
            dimension_semantics=("parallel",), vmem_limit_bytes=_VMEM),
    )(av, lip_conv_w, lip_conv_b, fclip_fc1_w, fclip_fc1_b)

    sa, ta = _bn_fold(final_bn_aud_gamma, final_bn_aud_beta,
                      final_bn_aud_rm, final_bn_aud_rv)
    sv, tv = _bn_fold(final_bn_lip_gamma, final_bn_lip_beta,
                      final_bn_lip_rm, final_bn_lip_rv)
    outs = pl.pallas_call(
        _heads_body,
        out_shape=(
            jax.ShapeDtypeStruct((B, 128), _F32),
            jax.ShapeDtypeStruct((B, 1024), _F32),
            jax.ShapeDtypeStruct((B, 1024), _F32),
            jax.ShapeDtypeStruct((B, 128), _F32),
            jax.ShapeDtypeStruct((B, 128), _F32),
        ),
        compiler_params=pltpu.CompilerParams(vmem_limit_bytes=_VMEM),
    )(ha, hv, fcaud_fc2_w, fcaud_fc2_b, fclip_fc2_w, fclip_fc2_b,
      sa, ta, sv, tv,
      _padw(final_fc_aud_w), jnp.pad(final_fc_aud_b, (0, 126))[None, :],
      _padw(final_fc_lip_w), jnp.pad(final_fc_lip_b, (0, 126))[None, :],
      final_cls_w1, final_cls_b1[None, :],
      jnp.pad(jnp.transpose(final_cls_w2), ((0, 0), (0, 126))),
      jnp.pad(final_cls_b2, (0, 126))[None, :])

    fo, vid_feat, aud_feat, vc, ac = outs
    return (fo[:, :2], vid_feat, aud_feat, vc[:, :2], ac[:, :2])
